# Initial kernel scaffold; baseline (speedup 1.0000x reference)
import jax
import jax.numpy as jnp
from jax.experimental import pallas as pl


def kernel(queries, clip_keys, frame_keys, gt_indices):
    raise NotImplementedError("write your pallas kernel here")


# speedup vs baseline: 148.8333x; 148.8333x over previous
import functools

import jax
import jax.numpy as jnp
from jax import lax
from jax.experimental import pallas as pl
from jax.experimental.pallas import tpu as pltpu
from jax.experimental.pallas import tpu_sc as plsc

W_CLIP = 0.7
W_FRAME = 0.3

Q, K, D = 4096, 16384, 512
BK = 512
KT = K // BK
NW = 32
BPW = Q // NW


def _l2n(x):
    return x / jnp.maximum(jnp.sqrt(jnp.sum(x * x, axis=1, keepdims=True)), 1e-12)


def _gather_gt_rows(clip_keys, frame_keys, gt_indices):
    mesh = plsc.VectorSubcoreMesh(core_axis_name="c", subcore_axis_name="s")

    @functools.partial(
        pl.kernel,
        mesh=mesh,
        out_type=[
            jax.ShapeDtypeStruct((Q, D), jnp.float32),
            jax.ShapeDtypeStruct((Q, D), jnp.float32),
        ],
        scratch_types=[
            pltpu.VMEM((BPW,), jnp.int32),
            pltpu.VMEM((BPW, D), jnp.float32),
            pltpu.SemaphoreType.DMA,
        ],
    )
    def gather_k(ck_hbm, fk_hbm, idx_hbm, gck_hbm, gfk_hbm, idx_v, rows_v, sem):
        wid = lax.axis_index("s") * 2 + lax.axis_index("c")
        base = wid * BPW
        pltpu.sync_copy(idx_hbm.at[pl.ds(base, BPW)], idx_v)
        pltpu.async_copy(ck_hbm.at[idx_v], rows_v, sem).wait()
        pltpu.sync_copy(rows_v, gck_hbm.at[pl.ds(base, BPW)])
        pltpu.async_copy(fk_hbm.at[idx_v], rows_v, sem).wait()
        pltpu.sync_copy(rows_v, gfk_hbm.at[pl.ds(base, BPW)])

    return gather_k(clip_keys, frame_keys, gt_indices)


def _prep_body(q_ref, gck_ref, gfk_ref, qn_ref, gts_ref):
    qn = _l2n(q_ref[...])
    qn_ref[...] = qn
    gcomb = W_CLIP * _l2n(gck_ref[...]) + W_FRAME * _l2n(gfk_ref[...])
    gts_ref[...] = jnp.sum(qn * gcomb, axis=1, keepdims=True)


def _prep(queries, gck, gfk):
    return pl.pallas_call(
        _prep_body,
        grid=(1,),
        in_specs=[
            pl.BlockSpec((Q, D), lambda i: (0, 0)),
            pl.BlockSpec((Q, D), lambda i: (0, 0)),
            pl.BlockSpec((Q, D), lambda i: (0, 0)),
        ],
        out_specs=[
            pl.BlockSpec((Q, D), lambda i: (0, 0)),
            pl.BlockSpec((Q, 1), lambda i: (0, 0)),
        ],
        out_shape=[
            jax.ShapeDtypeStruct((Q, D), jnp.float32),
            jax.ShapeDtypeStruct((Q, 1), jnp.float32),
        ],
    )(queries, gck, gfk)


def _main_body(qn_ref, ck_ref, fk_ref, gt_ref, gts_ref,
               score_ref, recalls_ref, cnt_ref):
    k = pl.program_id(0)

    @pl.when(k == 0)
    def _():
        cnt_ref[...] = jnp.zeros_like(cnt_ref)

    comb = W_CLIP * _l2n(ck_ref[...]) + W_FRAME * _l2n(fk_ref[...])
    s = lax.dot_general(qn_ref[...], comb, (((1,), (1,)), ((), ())),
                        preferred_element_type=jnp.float32)
    score_ref[...] = s

    cols = lax.broadcasted_iota(jnp.int32, (Q, BK), 1) + k * BK
    hits = jnp.where((s > gts_ref[...]) & (cols != gt_ref[...]), 1.0, 0.0)
    cnt_ref[...] += jnp.sum(hits, axis=1, keepdims=True)

    @pl.when(k == KT - 1)
    def _():
        rank = cnt_ref[...] + 1.0
        r1 = 100.0 / Q * jnp.sum(jnp.where(rank <= 1.0, 1.0, 0.0))
        r5 = 100.0 / Q * jnp.sum(jnp.where(rank <= 5.0, 1.0, 0.0))
        r10 = 100.0 / Q * jnp.sum(jnp.where(rank <= 10.0, 1.0, 0.0))
        r100 = 100.0 / Q * jnp.sum(jnp.where(rank <= 100.0, 1.0, 0.0))
        recalls_ref[0:1, :] = jnp.full((1, 128), r1, jnp.float32)
        recalls_ref[1:2, :] = jnp.full((1, 128), r5, jnp.float32)
        recalls_ref[2:3, :] = jnp.full((1, 128), r10, jnp.float32)
        recalls_ref[3:4, :] = jnp.full((1, 128), r100, jnp.float32)
        recalls_ref[4:5, :] = jnp.full((1, 128), r1 + r5 + r10 + r100,
                                       jnp.float32)
        recalls_ref[5:8, :] = jnp.zeros((3, 128), jnp.float32)


def _main(qn, clip_keys, frame_keys, gt2d, gts):
    return pl.pallas_call(
        _main_body,
        grid=(KT,),
        in_specs=[
            pl.BlockSpec((Q, D), lambda k: (0, 0)),
            pl.BlockSpec((BK, D), lambda k: (k, 0)),
            pl.BlockSpec((BK, D), lambda k: (k, 0)),
            pl.BlockSpec((Q, 1), lambda k: (0, 0)),
            pl.BlockSpec((Q, 1), lambda k: (0, 0)),
        ],
        out_specs=[
            pl.BlockSpec((Q, BK), lambda k: (0, k)),
            pl.BlockSpec((8, 128), lambda k: (0, 0)),
        ],
        out_shape=[
            jax.ShapeDtypeStruct((Q, K), jnp.float32),
            jax.ShapeDtypeStruct((8, 128), jnp.float32),
        ],
        scratch_shapes=[pltpu.VMEM((Q, 1), jnp.float32)],
    )(qn, clip_keys, frame_keys, gt2d, gts)


def kernel(queries, clip_keys, frame_keys, gt_indices):
    gck, gfk = _gather_gt_rows(clip_keys, frame_keys, gt_indices)
    qn, gts = _prep(queries, gck, gfk)
    gt2d = gt_indices.reshape(Q, 1)
    score, rec = _main(qn, clip_keys, frame_keys, gt2d, gts)
    return score, rec[:5, 0]


# speedup vs baseline: 149.5133x; 1.0046x over previous
import functools

import jax
import jax.numpy as jnp
from jax import lax
from jax.experimental import pallas as pl
from jax.experimental.pallas import tpu as pltpu
from jax.experimental.pallas import tpu_sc as plsc

W_CLIP = 0.7
W_FRAME = 0.3

Q, K, D = 4096, 16384, 512
BK = 512
KT = K // BK
NW = 32
BPW = Q // NW


def _l2n(x):
    return x / jnp.maximum(jnp.sqrt(jnp.sum(x * x, axis=1, keepdims=True)), 1e-12)


def _gather_gt_rows(clip_keys, frame_keys, gt_indices):
    mesh = plsc.VectorSubcoreMesh(core_axis_name="c", subcore_axis_name="s")

    @functools.partial(
        pl.kernel,
        mesh=mesh,
        out_type=[
            jax.ShapeDtypeStruct((Q, D), jnp.float32),
            jax.ShapeDtypeStruct((Q, D), jnp.float32),
        ],
        scratch_types=[
            pltpu.VMEM((BPW,), jnp.int32),
            pltpu.VMEM((BPW, D), jnp.float32),
            pltpu.SemaphoreType.DMA,
        ],
    )
    def gather_k(ck_hbm, fk_hbm, idx_hbm, gck_hbm, gfk_hbm, idx_v, rows_v, sem):
        wid = lax.axis_index("s") * 2 + lax.axis_index("c")
        base = wid * BPW
        pltpu.sync_copy(idx_hbm.at[pl.ds(base, BPW)], idx_v)
        pltpu.async_copy(ck_hbm.at[idx_v], rows_v, sem).wait()
        pltpu.sync_copy(rows_v, gck_hbm.at[pl.ds(base, BPW)])
        pltpu.async_copy(fk_hbm.at[idx_v], rows_v, sem).wait()
        pltpu.sync_copy(rows_v, gfk_hbm.at[pl.ds(base, BPW)])

    return gather_k(clip_keys, frame_keys, gt_indices)


BC = 512
NC_PREP = Q // BC


def _main_body(q_hbm, ck_ref, fk_ref, gt_ref, gck_hbm, gfk_hbm,
               score_ref, recalls_ref,
               qn_ref, gts_ref, cnt_ref, bufq, buf1, buf2,
               semq, sem1, sem2):
    k = pl.program_id(0)

    @pl.when(k == 0)
    def _():
        cnt_ref[...] = jnp.zeros_like(cnt_ref)
        for c in range(NC_PREP):
            cpq = pltpu.make_async_copy(
                q_hbm.at[pl.ds(c * BC, BC), :], bufq, semq)
            cp1 = pltpu.make_async_copy(
                gck_hbm.at[pl.ds(c * BC, BC), :], buf1, sem1)
            cp2 = pltpu.make_async_copy(
                gfk_hbm.at[pl.ds(c * BC, BC), :], buf2, sem2)
            cpq.start()
            cp1.start()
            cp2.start()
            cpq.wait()
            cp1.wait()
            cp2.wait()
            qn_c = _l2n(bufq[...])
            qn_ref[pl.ds(c * BC, BC), :] = qn_c
            gcomb = W_CLIP * _l2n(buf1[...]) + W_FRAME * _l2n(buf2[...])
            gts_ref[pl.ds(c * BC, BC), :] = jnp.sum(
                qn_c * gcomb, axis=1, keepdims=True)

    comb = W_CLIP * _l2n(ck_ref[...]) + W_FRAME * _l2n(fk_ref[...])
    s = lax.dot_general(qn_ref[...], comb, (((1,), (1,)), ((), ())),
                        preferred_element_type=jnp.float32)
    score_ref[...] = s

    li = gt_ref[...] - k * BK
    cols = lax.broadcasted_iota(jnp.int32, (Q, BK), 1)
    hits = jnp.where((s > gts_ref[...]) & (cols != li), 1.0, 0.0)
    cnt_ref[...] += jnp.sum(hits, axis=1, keepdims=True)

    @pl.when(k == KT - 1)
    def _():
        rank = cnt_ref[...] + 1.0
        r1 = 100.0 / Q * jnp.sum(jnp.where(rank <= 1.0, 1.0, 0.0))
        r5 = 100.0 / Q * jnp.sum(jnp.where(rank <= 5.0, 1.0, 0.0))
        r10 = 100.0 / Q * jnp.sum(jnp.where(rank <= 10.0, 1.0, 0.0))
        r100 = 100.0 / Q * jnp.sum(jnp.where(rank <= 100.0, 1.0, 0.0))
        recalls_ref[0:1, :] = jnp.full((1, 128), r1, jnp.float32)
        recalls_ref[1:2, :] = jnp.full((1, 128), r5, jnp.float32)
        recalls_ref[2:3, :] = jnp.full((1, 128), r10, jnp.float32)
        recalls_ref[3:4, :] = jnp.full((1, 128), r100, jnp.float32)
        recalls_ref[4:5, :] = jnp.full((1, 128), r1 + r5 + r10 + r100,
                                       jnp.float32)
        recalls_ref[5:8, :] = jnp.zeros((3, 128), jnp.float32)


def _main(queries, clip_keys, frame_keys, gt2d, gck, gfk):
    return pl.pallas_call(
        _main_body,
        grid=(KT,),
        in_specs=[
            pl.BlockSpec(memory_space=pl.ANY),
            pl.BlockSpec((BK, D), lambda k: (k, 0)),
            pl.BlockSpec((BK, D), lambda k: (k, 0)),
            pl.BlockSpec((Q, 1), lambda k: (0, 0)),
            pl.BlockSpec(memory_space=pl.ANY),
            pl.BlockSpec(memory_space=pl.ANY),
        ],
        out_specs=[
            pl.BlockSpec((Q, BK), lambda k: (0, k)),
            pl.BlockSpec((8, 128), lambda k: (0, 0)),
        ],
        out_shape=[
            jax.ShapeDtypeStruct((Q, K), jnp.float32),
            jax.ShapeDtypeStruct((8, 128), jnp.float32),
        ],
        scratch_shapes=[
            pltpu.VMEM((Q, D), jnp.float32),
            pltpu.VMEM((Q, 1), jnp.float32),
            pltpu.VMEM((Q, 1), jnp.float32),
            pltpu.VMEM((BC, D), jnp.float32),
            pltpu.VMEM((BC, D), jnp.float32),
            pltpu.VMEM((BC, D), jnp.float32),
            pltpu.SemaphoreType.DMA,
            pltpu.SemaphoreType.DMA,
            pltpu.SemaphoreType.DMA,
        ],
    )(queries, clip_keys, frame_keys, gt2d, gck, gfk)


def kernel(queries, clip_keys, frame_keys, gt_indices):
    gck, gfk = _gather_gt_rows(clip_keys, frame_keys, gt_indices)
    gt2d = gt_indices.reshape(Q, 1)
    score, rec = _main(queries, clip_keys, frame_keys, gt2d, gck, gfk)
    return score, rec[:5, 0]
